# baseline (speedup 1.0000x reference)
import functools

import jax
import jax.numpy as jnp
from jax import lax
from jax.experimental import pallas as pl
from jax.experimental.pallas import tpu as pltpu
from jax.experimental.pallas import tpu_sc as plsc

NC, NS, LANES = 2, 16, 16
NW = NC * NS

BATCH = 1024
SEQ = 200
DIM = 64
LINE = 2 * DIM
VOCAB = 1000000
HALF = SEQ // 2
IPAD = 112
ROWS_PER_W = BATCH // NW
EPS = 1e-5
NVR = DIM // LANES


def _body(ids_hbm, w_hbm, pos_hbm, g_hbm, b_hbm, out_hbm,
          idx_v, rows_v, pos_v, out_v, g_v, b_v, gsem, osem):
  wid = lax.axis_index("s") * NC + lax.axis_index("c")
  base_b = wid * ROWS_PER_W

  pltpu.sync_copy(pos_hbm, pos_v)
  pltpu.sync_copy(g_hbm, g_v)
  pltpu.sync_copy(b_hbm, b_v)
  g_regs = [g_v[pl.ds(LANES * j, LANES)] for j in range(NVR)]
  b_regs = [b_v[pl.ds(LANES * j, LANES)] for j in range(NVR)]

  lane = lax.iota(jnp.int32, LANES)
  gdn = lax.GatherDimensionNumbers(
      offset_dims=(), collapsed_slice_dims=(0,), start_index_map=(0,))

  def shuffle(x, perm):
    return lax.gather(x, perm[:, None], gdn, (1,),
                      mode=lax.GatherScatterMode.PROMISE_IN_BOUNDS)

  def hsum(x):
    for k in (8, 4, 2, 1):
      x = x + shuffle(x, lane ^ k)
    return x

  def compute(buf):
    def tok_body(t, _):
      s = [rows_v[buf, t, pl.ds(LANES * j, LANES)]
           + pos_v[t, pl.ds(LANES * j, LANES)] for j in range(NVR)]
      tot = (s[0] + s[1]) + (s[2] + s[3])
      sq = (s[0] * s[0] + s[1] * s[1]) + (s[2] * s[2] + s[3] * s[3])
      mv = hsum(tot) * (1.0 / DIM)
      av = hsum(sq) * (1.0 / DIM) - mv * mv + EPS
      i = lax.bitcast_convert_type(av, jnp.int32)
      y = lax.bitcast_convert_type(jnp.int32(0x5F3759DF) - (i >> 1),
                                   jnp.float32)
      half = av * 0.5
      for _ in range(3):
        y = y * (1.5 - half * y * y)
      for j in range(NVR):
        out_v[buf, t, pl.ds(LANES * j, LANES)] = \
            (s[j] - mv) * y * g_regs[j] + b_regs[j]
      return 0
    lax.fori_loop(0, SEQ, tok_body, 0)

  def stage_gather(r, buf):
    b_idx = base_b + r
    for h in (0, 1):
      pltpu.sync_copy(ids_hbm.at[b_idx, h], idx_v.at[buf, h, pl.ds(0, HALF)])
      pltpu.async_copy(w_hbm.at[idx_v.at[buf, h, pl.ds(0, HALF)]],
                       rows_v.at[buf, pl.ds(h * HALF, HALF)], gsem)

  def wait_gather(buf):
    for h in (0, 1):
      pltpu.make_async_copy(w_hbm.at[idx_v.at[buf, h, pl.ds(0, HALF)]],
                            rows_v.at[buf, pl.ds(h * HALF, HALF)], gsem).wait()

  def start_out(r, buf):
    pltpu.async_copy(out_v.at[buf], out_hbm.at[base_b + r], osem)

  def wait_out(buf):
    pltpu.make_async_copy(out_v.at[buf], out_hbm.at[base_b], osem).wait()

  stage_gather(0, 0)

  def pair_body(i, _):
    @pl.when(i > 0)
    def _():
      wait_out(1)
    stage_gather(2 * i + 1, 1)
    wait_gather(0)
    compute(0)
    start_out(2 * i, 0)

    wait_out(0)
    @pl.when(i + 1 < ROWS_PER_W // 2)
    def _():
      stage_gather(2 * i + 2, 0)
    wait_gather(1)
    compute(1)
    start_out(2 * i + 1, 1)
    return 0

  lax.fori_loop(0, ROWS_PER_W // 2, pair_body, 0)
  wait_out(1)


@functools.partial(
    pl.kernel,
    out_type=jax.ShapeDtypeStruct((BATCH, SEQ, DIM), jnp.float32),
    mesh=plsc.VectorSubcoreMesh(core_axis_name="c", subcore_axis_name="s"),
    scratch_types=[
        pltpu.VMEM((2, 2, IPAD), jnp.int32),
        pltpu.VMEM((2, SEQ, LINE), jnp.float32),
        pltpu.VMEM((SEQ, DIM), jnp.float32),
        pltpu.VMEM((2, SEQ, DIM), jnp.float32),
        pltpu.VMEM((DIM,), jnp.float32),
        pltpu.VMEM((DIM,), jnp.float32),
        pltpu.SemaphoreType.DMA,
        pltpu.SemaphoreType.DMA,
    ],
)
def _sc_embed(ids_hbm, w_hbm, pos_hbm, g_hbm, b_hbm, out_hbm,
              idx_v, rows_v, pos_v, out_v, g_v, b_v, gsem, osem):
  _body(ids_hbm, w_hbm, pos_hbm, g_hbm, b_hbm, out_hbm,
        idx_v, rows_v, pos_v, out_v, g_v, b_v, gsem, osem)


@jax.jit
def kernel(input_ids, W_word, pos_table, ln_gamma, ln_beta):
  ids = input_ids.reshape(BATCH, 2, HALF).astype(jnp.int32)
  w128 = jnp.concatenate([W_word, W_word], axis=1)
  pos_slice = pos_table[:SEQ]
  return _sc_embed(ids, w128, pos_slice, ln_gamma, ln_beta)


# speedup vs baseline: 1.1927x; 1.1927x over previous
import functools

import jax
import jax.numpy as jnp
from jax import lax
from jax.experimental import pallas as pl
from jax.experimental.pallas import tpu as pltpu
from jax.experimental.pallas import tpu_sc as plsc

NC, NS, LANES = 2, 16, 16
NW = NC * NS

BATCH = 1024
SEQ = 200
DIM = 64
LINE = 2 * DIM
VOCAB = 1000000
HALF = SEQ // 2
IPAD = 112
ROWS_PER_W = BATCH // NW
EPS = 1e-5
NVR = DIM // LANES


def _body(ids_hbm, w_hbm, pos_hbm, g_hbm, b_hbm, out_hbm,
          idx_v, rows_v, pos_v, out_v, g_v, b_v, gsem, osem):
  wid = lax.axis_index("s") * NC + lax.axis_index("c")
  base_b = wid * ROWS_PER_W

  pltpu.sync_copy(pos_hbm, pos_v)
  pltpu.sync_copy(g_hbm, g_v)
  pltpu.sync_copy(b_hbm, b_v)
  g_regs = [g_v[pl.ds(LANES * j, LANES)] for j in range(NVR)]
  b_regs = [b_v[pl.ds(LANES * j, LANES)] for j in range(NVR)]

  lane = lax.iota(jnp.int32, LANES)
  gdn = lax.GatherDimensionNumbers(
      offset_dims=(), collapsed_slice_dims=(0,), start_index_map=(0,))

  def shuffle(x, perm):
    return lax.gather(x, perm[:, None], gdn, (1,),
                      mode=lax.GatherScatterMode.PROMISE_IN_BOUNDS)

  def hsum(x):
    for k in (8, 4, 2, 1):
      x = x + shuffle(x, lane ^ k)
    return x

  def compute(buf):
    @plsc.parallel_loop(0, SEQ, 1, unroll=2)
    def tok_body(t):
      s = [rows_v[buf, t, pl.ds(LANES * j, LANES)]
           + pos_v[t, pl.ds(LANES * j, LANES)] for j in range(NVR)]
      tot = (s[0] + s[1]) + (s[2] + s[3])
      sq = (s[0] * s[0] + s[1] * s[1]) + (s[2] * s[2] + s[3] * s[3])
      mv = hsum(tot) * (1.0 / DIM)
      av = hsum(sq) * (1.0 / DIM) - mv * mv + EPS
      i = lax.bitcast_convert_type(av, jnp.int32)
      y = lax.bitcast_convert_type(jnp.int32(0x5F3759DF) - (i >> 1),
                                   jnp.float32)
      half = av * 0.5
      for _ in range(3):
        y = y * (1.5 - half * y * y)
      for j in range(NVR):
        out_v[buf, t, pl.ds(LANES * j, LANES)] = \
            (s[j] - mv) * y * g_regs[j] + b_regs[j]

  def stage_gather(r, buf):
    b_idx = base_b + r
    for h in (0, 1):
      pltpu.sync_copy(ids_hbm.at[b_idx, h], idx_v.at[buf, h, pl.ds(0, HALF)])
      pltpu.async_copy(w_hbm.at[idx_v.at[buf, h, pl.ds(0, HALF)]],
                       rows_v.at[buf, pl.ds(h * HALF, HALF)], gsem)

  def wait_gather(buf):
    for h in (0, 1):
      pltpu.make_async_copy(w_hbm.at[idx_v.at[buf, h, pl.ds(0, HALF)]],
                            rows_v.at[buf, pl.ds(h * HALF, HALF)], gsem).wait()

  def start_out(r, buf):
    pltpu.async_copy(out_v.at[buf], out_hbm.at[base_b + r], osem)

  def wait_out(buf):
    pltpu.make_async_copy(out_v.at[buf], out_hbm.at[base_b], osem).wait()

  stage_gather(0, 0)

  def pair_body(i, _):
    @pl.when(i > 0)
    def _():
      wait_out(1)
    stage_gather(2 * i + 1, 1)
    wait_gather(0)
    compute(0)
    start_out(2 * i, 0)

    wait_out(0)
    @pl.when(i + 1 < ROWS_PER_W // 2)
    def _():
      stage_gather(2 * i + 2, 0)
    wait_gather(1)
    compute(1)
    start_out(2 * i + 1, 1)
    return 0

  lax.fori_loop(0, ROWS_PER_W // 2, pair_body, 0)
  wait_out(1)


@functools.partial(
    pl.kernel,
    out_type=jax.ShapeDtypeStruct((BATCH, SEQ, DIM), jnp.float32),
    mesh=plsc.VectorSubcoreMesh(core_axis_name="c", subcore_axis_name="s"),
    scratch_types=[
        pltpu.VMEM((2, 2, IPAD), jnp.int32),
        pltpu.VMEM((2, SEQ, LINE), jnp.float32),
        pltpu.VMEM((SEQ, DIM), jnp.float32),
        pltpu.VMEM((2, SEQ, DIM), jnp.float32),
        pltpu.VMEM((DIM,), jnp.float32),
        pltpu.VMEM((DIM,), jnp.float32),
        pltpu.SemaphoreType.DMA,
        pltpu.SemaphoreType.DMA,
    ],
)
def _sc_embed(ids_hbm, w_hbm, pos_hbm, g_hbm, b_hbm, out_hbm,
              idx_v, rows_v, pos_v, out_v, g_v, b_v, gsem, osem):
  _body(ids_hbm, w_hbm, pos_hbm, g_hbm, b_hbm, out_hbm,
        idx_v, rows_v, pos_v, out_v, g_v, b_v, gsem, osem)


@jax.jit
def kernel(input_ids, W_word, pos_table, ln_gamma, ln_beta):
  ids = input_ids.reshape(BATCH, 2, HALF).astype(jnp.int32)
  w128 = jnp.pad(W_word, ((0, 0), (0, LINE - DIM)))
  pos_slice = pos_table[:SEQ]
  return _sc_embed(ids, w128, pos_slice, ln_gamma, ln_beta)


# speedup vs baseline: 1.2449x; 1.0437x over previous
import functools

import jax
import jax.numpy as jnp
from jax import lax
from jax.experimental import pallas as pl
from jax.experimental.pallas import tpu as pltpu
from jax.experimental.pallas import tpu_sc as plsc

NC, NS, LANES = 2, 16, 16
NW = NC * NS

BATCH = 1024
SEQ = 200
DIM = 64
LINE = 2 * DIM
VOCAB = 1000000
HALF = SEQ // 2
IPAD = 112
ROWS_PER_W = BATCH // NW
EPS = 1e-5
NVR = DIM // LANES


def _body(ids_hbm, w_hbm, pos_hbm, g_hbm, b_hbm, out_hbm,
          idx_v, rows_v, pos_v, out_v, g_v, b_v, gsem, osem):
  wid = lax.axis_index("s") * NC + lax.axis_index("c")
  base_b = wid * ROWS_PER_W

  pltpu.sync_copy(ids_hbm.at[pl.ds(base_b, ROWS_PER_W)], idx_v)
  pltpu.sync_copy(pos_hbm, pos_v)
  pltpu.sync_copy(g_hbm, g_v)
  pltpu.sync_copy(b_hbm, b_v)
  g_regs = [g_v[pl.ds(LANES * j, LANES)] for j in range(NVR)]
  b_regs = [b_v[pl.ds(LANES * j, LANES)] for j in range(NVR)]

  lane = lax.iota(jnp.int32, LANES)
  gdn = lax.GatherDimensionNumbers(
      offset_dims=(), collapsed_slice_dims=(0,), start_index_map=(0,))

  def shuffle(x, perm):
    return lax.gather(x, perm[:, None], gdn, (1,),
                      mode=lax.GatherScatterMode.PROMISE_IN_BOUNDS)

  def hsum(x):
    for k in (8, 4, 2, 1):
      x = x + shuffle(x, lane ^ k)
    return x

  def compute(buf):
    @plsc.parallel_loop(0, SEQ, 1, unroll=2)
    def tok_body(t):
      s = [rows_v[buf, t, pl.ds(LANES * j, LANES)]
           + pos_v[t, pl.ds(LANES * j, LANES)] for j in range(NVR)]
      tot = (s[0] + s[1]) + (s[2] + s[3])
      sq = (s[0] * s[0] + s[1] * s[1]) + (s[2] * s[2] + s[3] * s[3])
      mv = hsum(tot) * (1.0 / DIM)
      av = hsum(sq) * (1.0 / DIM) - mv * mv + EPS
      i = lax.bitcast_convert_type(av, jnp.int32)
      y = lax.bitcast_convert_type(jnp.int32(0x5F3759DF) - (i >> 1),
                                   jnp.float32)
      half = av * 0.5
      for _ in range(3):
        y = y * (1.5 - half * y * y)
      for j in range(NVR):
        out_v[t, pl.ds(LANES * j, LANES)] = \
            (s[j] - mv) * y * g_regs[j] + b_regs[j]

  def stage_gather(r, buf):
    for h in (0, 1):
      pltpu.async_copy(w_hbm.at[idx_v.at[r, h]],
                       rows_v.at[buf, pl.ds(h * HALF, HALF)], gsem)

  def wait_gather(buf):
    for h in (0, 1):
      pltpu.make_async_copy(w_hbm.at[idx_v.at[0, h]],
                            rows_v.at[buf, pl.ds(h * HALF, HALF)], gsem).wait()

  def start_out(r):
    pltpu.async_copy(out_v, out_hbm.at[base_b + r], osem)

  def wait_out():
    pltpu.make_async_copy(out_v, out_hbm.at[base_b], osem).wait()

  stage_gather(0, 0)

  def pair_body(i, _):
    stage_gather(2 * i + 1, 1)
    wait_gather(0)
    @pl.when(i > 0)
    def _():
      wait_out()
    compute(0)
    start_out(2 * i)

    @pl.when(i + 1 < ROWS_PER_W // 2)
    def _():
      stage_gather(2 * i + 2, 0)
    wait_gather(1)
    wait_out()
    compute(1)
    start_out(2 * i + 1)
    return 0

  lax.fori_loop(0, ROWS_PER_W // 2, pair_body, 0)
  wait_out()


@functools.partial(
    pl.kernel,
    out_type=jax.ShapeDtypeStruct((BATCH, SEQ, DIM), jnp.float32),
    mesh=plsc.VectorSubcoreMesh(core_axis_name="c", subcore_axis_name="s"),
    scratch_types=[
        pltpu.VMEM((ROWS_PER_W, 2, HALF), jnp.int32),
        pltpu.VMEM((2, SEQ, LINE), jnp.float32),
        pltpu.VMEM((SEQ, DIM), jnp.float32),
        pltpu.VMEM((SEQ, DIM), jnp.float32),
        pltpu.VMEM((DIM,), jnp.float32),
        pltpu.VMEM((DIM,), jnp.float32),
        pltpu.SemaphoreType.DMA,
        pltpu.SemaphoreType.DMA,
    ],
)
def _sc_embed(ids_hbm, w_hbm, pos_hbm, g_hbm, b_hbm, out_hbm,
              idx_v, rows_v, pos_v, out_v, g_v, b_v, gsem, osem):
  _body(ids_hbm, w_hbm, pos_hbm, g_hbm, b_hbm, out_hbm,
        idx_v, rows_v, pos_v, out_v, g_v, b_v, gsem, osem)


@jax.jit
def kernel(input_ids, W_word, pos_table, ln_gamma, ln_beta):
  ids = input_ids.reshape(BATCH, 2, HALF).astype(jnp.int32)
  w128 = jnp.pad(W_word, ((0, 0), (0, LINE - DIM)))
  pos_slice = pos_table[:SEQ]
  return _sc_embed(ids, w128, pos_slice, ln_gamma, ln_beta)


# speedup vs baseline: 1.2449x; 1.0000x over previous
import functools

import jax
import jax.numpy as jnp
from jax import lax
from jax.experimental import pallas as pl
from jax.experimental.pallas import tpu as pltpu
from jax.experimental.pallas import tpu_sc as plsc

NC, NS, LANES = 2, 16, 16
NW = NC * NS

BATCH = 1024
SEQ = 200
DIM = 64
LINE = 2 * DIM
VOCAB = 1000000
HALF = SEQ // 2
IPAD = 112
ROWS_PER_W = BATCH // NW
EPS = 1e-5
NVR = DIM // LANES


def _body(ids_hbm, w_hbm, pos_hbm, g_hbm, b_hbm, out_hbm,
          idx_v, rows_v, pos_v, out_v, g_v, b_v, gsem, osem):
  wid = lax.axis_index("s") * NC + lax.axis_index("c")
  base_b = wid * ROWS_PER_W

  pltpu.sync_copy(ids_hbm.at[pl.ds(base_b, ROWS_PER_W)], idx_v)
  pltpu.sync_copy(pos_hbm, pos_v)
  pltpu.sync_copy(g_hbm, g_v)
  pltpu.sync_copy(b_hbm, b_v)
  g_regs = [g_v[pl.ds(LANES * j, LANES)] for j in range(NVR)]
  b_regs = [b_v[pl.ds(LANES * j, LANES)] for j in range(NVR)]

  lane = lax.iota(jnp.int32, LANES)
  gdn = lax.GatherDimensionNumbers(
      offset_dims=(), collapsed_slice_dims=(0,), start_index_map=(0,))

  def shuffle(x, perm):
    return lax.gather(x, perm[:, None], gdn, (1,),
                      mode=lax.GatherScatterMode.PROMISE_IN_BOUNDS)

  def hsum(x):
    for k in (8, 4, 2, 1):
      x = x + shuffle(x, lane ^ k)
    return x

  def compute(buf):
    @plsc.parallel_loop(0, SEQ, 1, unroll=4)
    def tok_body(t):
      s = [rows_v[buf, t, pl.ds(LANES * j, LANES)]
           + pos_v[t, pl.ds(LANES * j, LANES)] for j in range(NVR)]
      tot = (s[0] + s[1]) + (s[2] + s[3])
      sq = (s[0] * s[0] + s[1] * s[1]) + (s[2] * s[2] + s[3] * s[3])
      mv = hsum(tot) * (1.0 / DIM)
      av = hsum(sq) * (1.0 / DIM) - mv * mv + EPS
      i = lax.bitcast_convert_type(av, jnp.int32)
      y = lax.bitcast_convert_type(jnp.int32(0x5F3759DF) - (i >> 1),
                                   jnp.float32)
      half = av * 0.5
      for _ in range(3):
        y = y * (1.5 - half * y * y)
      for j in range(NVR):
        out_v[t, pl.ds(LANES * j, LANES)] = \
            (s[j] - mv) * y * g_regs[j] + b_regs[j]

  def stage_gather(r, buf):
    for h in (0, 1):
      pltpu.async_copy(w_hbm.at[idx_v.at[r, h]],
                       rows_v.at[buf, pl.ds(h * HALF, HALF)], gsem)

  def wait_gather(buf):
    for h in (0, 1):
      pltpu.make_async_copy(w_hbm.at[idx_v.at[0, h]],
                            rows_v.at[buf, pl.ds(h * HALF, HALF)], gsem).wait()

  def start_out(r):
    pltpu.async_copy(out_v, out_hbm.at[base_b + r], osem)

  def wait_out():
    pltpu.make_async_copy(out_v, out_hbm.at[base_b], osem).wait()

  stage_gather(0, 0)

  def pair_body(i, _):
    stage_gather(2 * i + 1, 1)
    wait_gather(0)
    @pl.when(i > 0)
    def _():
      wait_out()
    compute(0)
    start_out(2 * i)

    @pl.when(i + 1 < ROWS_PER_W // 2)
    def _():
      stage_gather(2 * i + 2, 0)
    wait_gather(1)
    wait_out()
    compute(1)
    start_out(2 * i + 1)
    return 0

  lax.fori_loop(0, ROWS_PER_W // 2, pair_body, 0)
  wait_out()


@functools.partial(
    pl.kernel,
    out_type=jax.ShapeDtypeStruct((BATCH, SEQ, DIM), jnp.float32),
    mesh=plsc.VectorSubcoreMesh(core_axis_name="c", subcore_axis_name="s"),
    scratch_types=[
        pltpu.VMEM((ROWS_PER_W, 2, HALF), jnp.int32),
        pltpu.VMEM((2, SEQ, LINE), jnp.float32),
        pltpu.VMEM((SEQ, DIM), jnp.float32),
        pltpu.VMEM((SEQ, DIM), jnp.float32),
        pltpu.VMEM((DIM,), jnp.float32),
        pltpu.VMEM((DIM,), jnp.float32),
        pltpu.SemaphoreType.DMA,
        pltpu.SemaphoreType.DMA,
    ],
)
def _sc_embed(ids_hbm, w_hbm, pos_hbm, g_hbm, b_hbm, out_hbm,
              idx_v, rows_v, pos_v, out_v, g_v, b_v, gsem, osem):
  _body(ids_hbm, w_hbm, pos_hbm, g_hbm, b_hbm, out_hbm,
        idx_v, rows_v, pos_v, out_v, g_v, b_v, gsem, osem)


@jax.jit
def kernel(input_ids, W_word, pos_table, ln_gamma, ln_beta):
  ids = input_ids.reshape(BATCH, 2, HALF).astype(jnp.int32)
  w128 = jnp.pad(W_word, ((0, 0), (0, LINE - DIM)))
  pos_slice = pos_table[:SEQ]
  return _sc_embed(ids, w128, pos_slice, ln_gamma, ln_beta)


# speedup vs baseline: 1.2451x; 1.0002x over previous
import functools

import jax
import jax.numpy as jnp
from jax import lax
from jax.experimental import pallas as pl
from jax.experimental.pallas import tpu as pltpu
from jax.experimental.pallas import tpu_sc as plsc

NC, NS, LANES = 2, 16, 16
NW = NC * NS

BATCH = 1024
SEQ = 200
DIM = 64
LINE = 2 * DIM
VOCAB = 1000000
HALF = SEQ // 2
IPAD = 112
ROWS_PER_W = BATCH // NW
EPS = 1e-5
NVR = DIM // LANES


def _body(ids_hbm, w_hbm, pos_hbm, g_hbm, b_hbm, out_hbm,
          idx_v, rows_v, pos_v, out_v, g_v, b_v, gsem, osem):
  wid = lax.axis_index("s") * NC + lax.axis_index("c")
  base_b = wid * ROWS_PER_W

  pltpu.sync_copy(ids_hbm.at[pl.ds(base_b, ROWS_PER_W)], idx_v)
  pltpu.sync_copy(pos_hbm, pos_v)
  pltpu.sync_copy(g_hbm, g_v)
  pltpu.sync_copy(b_hbm, b_v)
  g_regs = [g_v[pl.ds(LANES * j, LANES)] for j in range(NVR)]
  b_regs = [b_v[pl.ds(LANES * j, LANES)] for j in range(NVR)]

  lane = lax.iota(jnp.int32, LANES)
  gdn = lax.GatherDimensionNumbers(
      offset_dims=(), collapsed_slice_dims=(0,), start_index_map=(0,))

  def shuffle(x, perm):
    return lax.gather(x, perm[:, None], gdn, (1,),
                      mode=lax.GatherScatterMode.PROMISE_IN_BOUNDS)

  def hsum(x):
    for k in (8, 4, 2, 1):
      x = x + shuffle(x, lane ^ k)
    return x

  def compute(buf):
    @plsc.parallel_loop(0, SEQ, 1, unroll=4)
    def tok_body(t):
      s = [rows_v[buf, t, pl.ds(LANES * j, LANES)]
           + pos_v[t, pl.ds(LANES * j, LANES)] for j in range(NVR)]
      tot = (s[0] + s[1]) + (s[2] + s[3])
      sq = (s[0] * s[0] + s[1] * s[1]) + (s[2] * s[2] + s[3] * s[3])
      mv = hsum(tot) * (1.0 / DIM)
      av = hsum(sq) * (1.0 / DIM) - mv * mv + EPS
      i = lax.bitcast_convert_type(av, jnp.int32)
      y = lax.bitcast_convert_type(jnp.int32(0x5F3759DF) - (i >> 1),
                                   jnp.float32)
      half = av * 0.5
      for _ in range(3):
        y = y * (1.5 - half * y * y)
      for j in range(NVR):
        out_v[t, pl.ds(LANES * j, LANES)] = \
            (s[j] - mv) * y * g_regs[j] + b_regs[j]

  def stage_gather(r, buf):
    for h in (0, 1):
      pltpu.async_copy(w_hbm.at[idx_v.at[r, h]],
                       rows_v.at[buf, pl.ds(h * HALF, HALF)], gsem)

  def wait_gather(buf):
    for h in (0, 1):
      pltpu.make_async_copy(w_hbm.at[idx_v.at[0, h]],
                            rows_v.at[buf, pl.ds(h * HALF, HALF)], gsem).wait()

  def start_out(r):
    pltpu.async_copy(out_v, out_hbm.at[base_b + r], osem)

  def wait_out():
    pltpu.make_async_copy(out_v, out_hbm.at[base_b], osem).wait()

  stage_gather(0, 0)

  def pair_body(i, _):
    stage_gather(2 * i + 1, 1)
    wait_gather(0)
    @pl.when(i > 0)
    def _():
      wait_out()
    compute(0)
    start_out(2 * i)

    @pl.when(i + 1 < ROWS_PER_W // 2)
    def _():
      stage_gather(2 * i + 2, 0)
    wait_gather(1)
    wait_out()
    compute(1)
    start_out(2 * i + 1)
    return 0

  lax.fori_loop(0, ROWS_PER_W // 2, pair_body, 0)
  wait_out()


@functools.partial(
    pl.kernel,
    out_type=jax.ShapeDtypeStruct((BATCH, SEQ, DIM), jnp.float32),
    mesh=plsc.VectorSubcoreMesh(core_axis_name="c", subcore_axis_name="s"),
    scratch_types=[
        pltpu.VMEM((ROWS_PER_W, 2, HALF), jnp.int32),
        pltpu.VMEM((2, SEQ, LINE), jnp.float32),
        pltpu.VMEM((SEQ, DIM), jnp.float32),
        pltpu.VMEM((SEQ, DIM), jnp.float32),
        pltpu.VMEM((DIM,), jnp.float32),
        pltpu.VMEM((DIM,), jnp.float32),
        pltpu.SemaphoreType.DMA,
        pltpu.SemaphoreType.DMA,
    ],
)
def _sc_embed(ids_hbm, w_hbm, pos_hbm, g_hbm, b_hbm, out_hbm,
              idx_v, rows_v, pos_v, out_v, g_v, b_v, gsem, osem):
  _body(ids_hbm, w_hbm, pos_hbm, g_hbm, b_hbm, out_hbm,
        idx_v, rows_v, pos_v, out_v, g_v, b_v, gsem, osem)


@jax.jit
def kernel(input_ids, W_word, pos_table, ln_gamma, ln_beta):
  ids = input_ids.reshape(BATCH, 2, HALF).astype(jnp.int32)
  w128 = lax.dynamic_update_slice(
      jnp.zeros((VOCAB, LINE), jnp.float32), W_word, (0, 0))
  pos_slice = pos_table[:SEQ]
  return _sc_embed(ids, w128, pos_slice, ln_gamma, ln_beta)


# speedup vs baseline: 1.2476x; 1.0020x over previous
import functools

import jax
import jax.numpy as jnp
from jax import lax
from jax.experimental import pallas as pl
from jax.experimental.pallas import tpu as pltpu
from jax.experimental.pallas import tpu_sc as plsc

NC, NS, LANES = 2, 16, 16
NW = NC * NS

BATCH = 1024
SEQ = 200
DIM = 64
LINE = 2 * DIM
VOCAB = 1000000
HALF = SEQ // 2
IPAD = 112
ROWS_PER_W = BATCH // NW
EPS = 1e-5
NVR = DIM // LANES


def _body(ids_hbm, w_hbm, pos_hbm, g_hbm, b_hbm, out_hbm,
          idx_v, rows_v, pos_v, out_v, g_v, b_v, gsem, osem):
  wid = lax.axis_index("s") * NC + lax.axis_index("c")
  base_b = wid * ROWS_PER_W

  pltpu.sync_copy(ids_hbm.at[pl.ds(base_b, ROWS_PER_W)], idx_v)
  pltpu.sync_copy(pos_hbm, pos_v)
  pltpu.sync_copy(g_hbm, g_v)
  pltpu.sync_copy(b_hbm, b_v)
  g_regs = [g_v[pl.ds(LANES * j, LANES)] for j in range(NVR)]
  b_regs = [b_v[pl.ds(LANES * j, LANES)] for j in range(NVR)]

  lane = lax.iota(jnp.int32, LANES)
  gdn = lax.GatherDimensionNumbers(
      offset_dims=(), collapsed_slice_dims=(0,), start_index_map=(0,))

  def shuffle(x, perm):
    return lax.gather(x, perm[:, None], gdn, (1,),
                      mode=lax.GatherScatterMode.PROMISE_IN_BOUNDS)

  def hsum(x):
    for k in (8, 4, 2, 1):
      x = x + shuffle(x, lane ^ k)
    return x

  def compute(buf):
    @plsc.parallel_loop(0, SEQ, 1, unroll=4)
    def tok_body(t):
      s = [rows_v[buf, t, pl.ds(LANES * j, LANES)]
           + pos_v[t, pl.ds(LANES * j, LANES)] for j in range(NVR)]
      tot = (s[0] + s[1]) + (s[2] + s[3])
      sq = (s[0] * s[0] + s[1] * s[1]) + (s[2] * s[2] + s[3] * s[3])
      mv = hsum(tot) * (1.0 / DIM)
      av = hsum(sq) * (1.0 / DIM) - mv * mv + EPS
      i = lax.bitcast_convert_type(av, jnp.int32)
      y = lax.bitcast_convert_type(jnp.int32(0x5F3759DF) - (i >> 1),
                                   jnp.float32)
      half = av * 0.5
      for _ in range(3):
        y = y * (1.5 - half * y * y)
      for j in range(NVR):
        out_v[t, pl.ds(LANES * j, LANES)] = \
            (s[j] - mv) * y * g_regs[j] + b_regs[j]

  def stage_gather(r, buf):
    for h in (0, 1):
      pltpu.async_copy(w_hbm.at[idx_v.at[r, h]],
                       rows_v.at[buf, pl.ds(h * HALF, HALF)], gsem)

  def wait_gather(buf):
    for h in (0, 1):
      pltpu.make_async_copy(w_hbm.at[idx_v.at[0, h]],
                            rows_v.at[buf, pl.ds(h * HALF, HALF)], gsem).wait()

  def start_out(r):
    pltpu.async_copy(out_v, out_hbm.at[base_b + r], osem)

  def wait_out():
    pltpu.make_async_copy(out_v, out_hbm.at[base_b], osem).wait()

  stage_gather(0, 0)

  def pair_body(i, _):
    stage_gather(2 * i + 1, 1)
    wait_gather(0)
    @pl.when(i > 0)
    def _():
      wait_out()
    compute(0)
    start_out(2 * i)

    @pl.when(i + 1 < ROWS_PER_W // 2)
    def _():
      stage_gather(2 * i + 2, 0)
    wait_gather(1)
    wait_out()
    compute(1)
    start_out(2 * i + 1)
    return 0

  lax.fori_loop(0, ROWS_PER_W // 2, pair_body, 0)
  wait_out()


@functools.partial(
    pl.kernel,
    out_type=jax.ShapeDtypeStruct((BATCH, SEQ, DIM), jnp.float32),
    mesh=plsc.VectorSubcoreMesh(core_axis_name="c", subcore_axis_name="s"),
    scratch_types=[
        pltpu.VMEM((ROWS_PER_W, 2, HALF), jnp.int32),
        pltpu.VMEM((2, SEQ, LINE), jnp.float32),
        pltpu.VMEM((SEQ, DIM), jnp.float32),
        pltpu.VMEM((SEQ, DIM), jnp.float32),
        pltpu.VMEM((DIM,), jnp.float32),
        pltpu.VMEM((DIM,), jnp.float32),
        pltpu.SemaphoreType.DMA,
        pltpu.SemaphoreType.DMA,
    ],
)
def _sc_embed(ids_hbm, w_hbm, pos_hbm, g_hbm, b_hbm, out_hbm,
              idx_v, rows_v, pos_v, out_v, g_v, b_v, gsem, osem):
  _body(ids_hbm, w_hbm, pos_hbm, g_hbm, b_hbm, out_hbm,
        idx_v, rows_v, pos_v, out_v, g_v, b_v, gsem, osem)


@jax.jit
def kernel(input_ids, W_word, pos_table, ln_gamma, ln_beta):
  ids = input_ids.reshape(BATCH, 2, HALF).astype(jnp.int32)
  w128 = jnp.pad(W_word, ((0, 0), (0, LINE - DIM)))
  pos_slice = pos_table[:SEQ]
  return _sc_embed(ids, w128, pos_slice, ln_gamma, ln_beta)
